# baseline (speedup 1.0000x reference)
import functools

import jax
import jax.numpy as jnp
from jax import lax
from jax.experimental import pallas as pl
from jax.experimental.pallas import tpu as pltpu
from jax.experimental.pallas import tpu_sc as plsc

N = 50000
D = 256
NUM_GRAPHS = 64
BLK = 512
NBLK = 98
GRP = 16
N_G = (NBLK * BLK) // GRP
CHUNKB = 96
LANES = 16
NCOLG = D // LANES
UNROLL = 8
NEG = float("-inf")


def _pass_a_body(h_ref, batch_ref, gmax_ref, sum_ref, cnt_ref,
                 acc_sum, acc_cnt):
    b = pl.program_id(0)

    @pl.when(b == 0)
    def _():
        acc_sum[...] = jnp.zeros((NUM_GRAPHS, D), jnp.float32)
        acc_cnt[...] = jnp.zeros((NUM_GRAPHS, 128), jnp.float32)

    x = h_ref[...]
    rid = lax.broadcasted_iota(jnp.int32, (BLK, 1), 0) + b * BLK
    valid = rid < N
    xs = jnp.where(valid, x, 0.0)
    xm = jnp.where(valid, x, NEG)

    ids = batch_ref[...].reshape(1, BLK)
    giota = lax.broadcasted_iota(jnp.int32, (NUM_GRAPHS, BLK), 0)
    onehot = (giota == ids).astype(jnp.float32)

    acc_sum[...] += lax.dot_general(
        onehot, xs, (((1,), (0,)), ((), ())),
        preferred_element_type=jnp.float32)
    acc_cnt[...] += jnp.broadcast_to(
        jnp.sum(onehot, axis=1, keepdims=True), (NUM_GRAPHS, 128))

    gmax_ref[...] = jnp.max(xm.reshape(BLK // GRP, GRP, D), axis=1)

    @pl.when(b == NBLK - 1)
    def _():
        sum_ref[...] = acc_sum[...]
        cnt_ref[...] = acc_cnt[...]


def _pass_a(h, batch_p):
    return pl.pallas_call(
        _pass_a_body,
        grid=(NBLK,),
        in_specs=[
            pl.BlockSpec((BLK, D), lambda b: (b, 0)),
            pl.BlockSpec((1, 1, BLK), lambda b: (b, 0, 0)),
        ],
        out_specs=[
            pl.BlockSpec((BLK // GRP, D), lambda b: (b, 0)),
            pl.BlockSpec((NUM_GRAPHS, D), lambda b: (0, 0)),
            pl.BlockSpec((NUM_GRAPHS, 128), lambda b: (0, 0)),
        ],
        out_shape=[
            jax.ShapeDtypeStruct((N_G, D), jnp.float32),
            jax.ShapeDtypeStruct((NUM_GRAPHS, D), jnp.float32),
            jax.ShapeDtypeStruct((NUM_GRAPHS, 128), jnp.float32),
        ],
        scratch_shapes=[
            pltpu.VMEM((NUM_GRAPHS, D), jnp.float32),
            pltpu.VMEM((NUM_GRAPHS, 128), jnp.float32),
        ],
    )(h, batch_p)


def _cnt_at(cnt_v, k):
    v = cnt_v[k, pl.ds(0, LANES)]
    return v[0].astype(jnp.int32)


def _winb(glo_al, c):
    return pl.multiple_of(jnp.minimum(glo_al + c * CHUNKB, N_G - CHUNKB), 8)


def _sc_max_body(h_hbm, gmax_hbm, cnt_hbm, max_hbm,
                 cnt_v, buf0, buf1, buf2, buf3, hbuf, max_st,
                 sem0, sem1, sem2, sem3, osem):
    wid = lax.axis_index("s") * 2 + lax.axis_index("c")
    pltpu.sync_copy(cnt_hbm, cnt_v)
    bufs = ((buf0, buf1), (buf2, buf3))
    sems = ((sem0, sem1), (sem2, sem3))

    g0 = wid * 2
    lo0 = lax.fori_loop(0, g0, lambda k, a: a + _cnt_at(cnt_v, k),
                        jnp.int32(0))
    n0 = _cnt_at(cnt_v, g0)
    n1 = _cnt_at(cnt_v, g0 + 1)
    ranges = ((g0, lo0, lo0 + n0), (g0 + 1, lo0 + n0, lo0 + n0 + n1))

    params = []
    for gi in range(2):
        g, lo, hi = ranges[gi]
        glo = (lo + GRP - 1) // GRP
        ghi = hi // GRP
        gend = jnp.maximum(ghi, glo)
        glo_al = (glo // 8) * 8
        num_chunks = jnp.maximum((gend - glo_al + CHUNKB - 1) // CHUNKB, 1)
        m = ((num_chunks + 1) // 2) * 2
        params.append((g, lo, hi, glo, ghi, gend, glo_al, m))
        for parity in range(2):
            pltpu.async_copy(gmax_hbm.at[pl.ds(_winb(glo_al, parity), CHUNKB)],
                             bufs[gi][parity], sems[gi][parity])

    for gi in range(2):
        g, lo, hi, glo, ghi, gend, glo_al, m = params[gi]

        def pair_body(c2, carry, lo=lo, glo=glo, gend=gend, glo_al=glo_al,
                      m=m, gi=gi):
            for parity in range(2):
                c = c2 * 2 + parity
                bf = bufs[gi][parity]
                w = _winb(glo_al, c)
                pltpu.make_async_copy(
                    gmax_hbm.at[pl.ds(w, CHUNKB)], bf, sems[gi][parity]).wait()
                base = glo_al + c * CHUNKB
                r_start = jnp.clip(jnp.maximum(glo, base) - w, 0, CHUNKB)
                r_end = jnp.clip(jnp.minimum(gend, base + CHUNKB) - w,
                                 0, CHUNKB)
                r_end = jnp.maximum(r_start, r_end)
                nu = (r_end - r_start) // UNROLL

                def rows_at(r0, k, maxs2, bf=bf):
                    new_m = list(maxs2)
                    for rr in range(k):
                        for j in range(NCOLG):
                            v = bf[r0 + rr, pl.ds(j * LANES, LANES)]
                            new_m[j] = jnp.maximum(new_m[j], v)
                    return tuple(new_m)

                def blk_body(i, carry2, rs=r_start):
                    return rows_at(rs + i * UNROLL, UNROLL, carry2)

                def row_body(r, carry2):
                    return rows_at(r, 1, carry2)

                carry = lax.fori_loop(0, nu, blk_body, carry)
                carry = lax.fori_loop(r_start + nu * UNROLL, r_end, row_body,
                                      carry)

                @pl.when(c + 2 < m)
                def _(c=c, bf=bf, parity=parity, glo_al=glo_al, gi=gi):
                    pltpu.async_copy(
                        gmax_hbm.at[pl.ds(_winb(glo_al, c + 2), CHUNKB)], bf,
                        sems[gi][parity])
            return carry

        init = tuple(jnp.full((LANES,), NEG, jnp.float32)
                     for _ in range(NCOLG))
        maxs = lax.fori_loop(0, m // 2, pair_body, init)

        head_end = jnp.minimum(glo * GRP, hi)
        tail_start = jnp.maximum(ghi * GRP, head_end)
        for (a, b_) in ((lo, head_end), (tail_start, hi)):
            w = pl.multiple_of(
                jnp.minimum((a // GRP) * GRP, N - GRP), 8)
            pltpu.sync_copy(h_hbm.at[pl.ds(w, GRP)], hbuf)
            r_s = jnp.clip(a - w, 0, GRP)
            r_e = jnp.clip(b_ - w, 0, GRP)
            r_e = jnp.maximum(r_s, r_e)

            def brow(r, maxs2):
                new_m = list(maxs2)
                for j in range(NCOLG):
                    v = hbuf[r, pl.ds(j * LANES, LANES)]
                    new_m[j] = jnp.maximum(new_m[j], v)
                return tuple(new_m)

            maxs = lax.fori_loop(r_s, r_e, brow, maxs)

        for j in range(NCOLG):
            max_st[gi, pl.ds(j * LANES, LANES)] = maxs[j]
        pltpu.async_copy(max_st.at[pl.ds(gi, 1)], max_hbm.at[pl.ds(g, 1)],
                         osem)

    for gi in range(2):
        g = params[gi][0]
        pltpu.make_async_copy(max_st.at[pl.ds(gi, 1)],
                              max_hbm.at[pl.ds(g, 1)], osem).wait()


def _sc_max(h, gmax, cnt128):
    mesh = plsc.VectorSubcoreMesh(core_axis_name="c", subcore_axis_name="s")
    f = pl.kernel(
        _sc_max_body,
        mesh=mesh,
        out_type=jax.ShapeDtypeStruct((NUM_GRAPHS, D), jnp.float32),
        scratch_types=[
            pltpu.VMEM((NUM_GRAPHS, 128), jnp.float32),
            pltpu.VMEM((CHUNKB, D), jnp.float32),
            pltpu.VMEM((CHUNKB, D), jnp.float32),
            pltpu.VMEM((CHUNKB, D), jnp.float32),
            pltpu.VMEM((CHUNKB, D), jnp.float32),
            pltpu.VMEM((GRP, D), jnp.float32),
            pltpu.VMEM((2, D), jnp.float32),
            pltpu.SemaphoreType.DMA,
            pltpu.SemaphoreType.DMA,
            pltpu.SemaphoreType.DMA,
            pltpu.SemaphoreType.DMA,
            pltpu.SemaphoreType.DMA,
        ],
    )
    return f(h, gmax, cnt128)


def _mlp_body(cnt_ref, sum_ref, max_ref, w1_ref, b1_ref, w2_ref, b2_ref,
              out_ref):
    cnt = jnp.maximum(cnt_ref[:, 0:1], 1.0)
    h_sum = sum_ref[...]
    h_max = max_ref[...]
    h_mean = h_sum / cnt
    x = jnp.concatenate([h_sum, h_mean, h_max], axis=1)
    y = lax.dot_general(x, w1_ref[...], (((1,), (1,)), ((), ())),
                        preferred_element_type=jnp.float32) + b1_ref[...]
    y = 0.5 * y * (1.0 + lax.erf(y * 0.7071067811865476))
    out_ref[...] = lax.dot_general(y, w2_ref[...], (((1,), (1,)), ((), ())),
                                   preferred_element_type=jnp.float32) + b2_ref[...]


def _mlp(cnt, h_sum, h_max, W1, b1, W2, b2):
    return pl.pallas_call(
        _mlp_body,
        out_shape=jax.ShapeDtypeStruct((NUM_GRAPHS, D), jnp.float32),
    )(cnt, h_sum, h_max, W1, b1.reshape(1, D), W2, b2.reshape(1, D))


def kernel(h, batch, W1, b1, W2, b2):
    batch_p = jnp.concatenate(
        [batch.astype(jnp.int32),
         jnp.full((NBLK * BLK - N,), NUM_GRAPHS, jnp.int32)]).reshape(
             NBLK, 1, BLK)
    gmax, h_sum, cnt128 = _pass_a(h, batch_p)
    h_max = _sc_max(h, gmax, cnt128)
    return _mlp(cnt128, h_sum, h_max, W1, b1, W2, b2)


# speedup vs baseline: 1.9033x; 1.9033x over previous
import functools

import jax
import jax.numpy as jnp
from jax import lax
from jax.experimental import pallas as pl
from jax.experimental.pallas import tpu as pltpu
from jax.experimental.pallas import tpu_sc as plsc

N = 50000
D = 256
NUM_GRAPHS = 64
CHUNK = 120
LANES = 16
NCOLG = D // LANES
N_PAD = 50048
UNROLL = 8


def _scalar_at(ref, idx):
    v = ref[0, pl.ds(idx, LANES)]
    return v[0]


def _starts_body(batch_ref, starts_ref):
    b = batch_ref[...]
    lane = lax.broadcasted_iota(jnp.int32, (1, 128), 1)
    acc = jnp.zeros((1, 128), jnp.int32)
    for g in range(NUM_GRAPHS + 1):
        c = jnp.sum((b < g).astype(jnp.int32))
        acc = acc + jnp.where(lane == g, c, 0)
    starts_ref[...] = acc


def _starts(batch):
    batch_p = jnp.concatenate(
        [batch.astype(jnp.int32),
         jnp.full((N_PAD - N,), NUM_GRAPHS, jnp.int32)]).reshape(
             N_PAD // 128, 128)
    return pl.pallas_call(
        _starts_body,
        out_shape=jax.ShapeDtypeStruct((1, 128), jnp.int32),
    )(batch_p)


def _win(lo_al, c):
    return pl.multiple_of(jnp.minimum(lo_al + c * CHUNK, N - CHUNK), 8)


def _sc_pool_body(h_hbm, starts_hbm, sum_hbm, max_hbm, cnt_hbm,
                  starts_v, buf0, buf1, buf2, buf3, sum_st, max_st, cnt_st,
                  sem0, sem1, sem2, sem3, osem):
    wid = lax.axis_index("s") * 2 + lax.axis_index("c")
    pltpu.sync_copy(starts_hbm, starts_v)
    bufs = ((buf0, buf1), (buf2, buf3))
    sems = ((sem0, sem1), (sem2, sem3))

    params = []
    for gi in range(2):
        g = wid * 2 + gi
        lo = _scalar_at(starts_v, g)
        hi = _scalar_at(starts_v, g + 1)
        lo_al = (lo // 8) * 8
        num_chunks = jnp.maximum((hi - lo_al + CHUNK - 1) // CHUNK, 1)
        m = ((num_chunks + 1) // 2) * 2
        params.append((g, lo, hi, lo_al, m))
        for parity in range(2):
            pltpu.async_copy(h_hbm.at[pl.ds(_win(lo_al, parity), CHUNK)],
                             bufs[gi][parity], sems[gi][parity])

    for gi in range(2):
        g, lo, hi, lo_al, m = params[gi]
        n = hi - lo

        def pair_body(c2, carry, lo=lo, hi=hi, lo_al=lo_al, m=m, gi=gi):
            for parity in range(2):
                c = c2 * 2 + parity
                bf = bufs[gi][parity]
                w = _win(lo_al, c)
                pltpu.make_async_copy(
                    h_hbm.at[pl.ds(w, CHUNK)], bf, sems[gi][parity]).wait()
                base = lo_al + c * CHUNK
                r_start = jnp.clip(jnp.maximum(lo, base) - w, 0, CHUNK)
                r_end = jnp.clip(jnp.minimum(hi, base + CHUNK) - w, 0, CHUNK)
                r_end = jnp.maximum(r_start, r_end)
                nu = (r_end - r_start) // UNROLL

                def rows_at(r0, k, carry2, bf=bf):
                    sums2, maxs2 = carry2
                    new_s = list(sums2)
                    new_m = list(maxs2)
                    for j in range(NCOLG):
                        for rr in range(k):
                            v = bf[r0 + rr, pl.ds(j * LANES, LANES)]
                            new_s[j] = new_s[j] + v
                            new_m[j] = jnp.maximum(new_m[j], v)
                    return (tuple(new_s), tuple(new_m))

                def blk_body(i, carry2, rs=r_start):
                    return rows_at(rs + i * UNROLL, UNROLL, carry2)

                def row_body(r, carry2):
                    return rows_at(r, 1, carry2)

                carry = lax.fori_loop(0, nu, blk_body, carry)
                carry = lax.fori_loop(r_start + nu * UNROLL, r_end, row_body,
                                      carry)

                @pl.when(c + 2 < m)
                def _(c=c, bf=bf, parity=parity, lo_al=lo_al, gi=gi):
                    pltpu.async_copy(h_hbm.at[pl.ds(_win(lo_al, c + 2), CHUNK)],
                                     bf, sems[gi][parity])
            return carry

        init = (tuple(jnp.zeros((LANES,), jnp.float32) for _ in range(NCOLG)),
                tuple(jnp.full((LANES,), -jnp.inf, jnp.float32)
                      for _ in range(NCOLG)))
        sums, maxs = lax.fori_loop(0, m // 2, pair_body, init)

        for j in range(NCOLG):
            sum_st[gi, pl.ds(j * LANES, LANES)] = sums[j]
            max_st[gi, pl.ds(j * LANES, LANES)] = maxs[j]
        cnt_st[gi, pl.ds(0, LANES)] = (jnp.full((LANES,), 1.0, jnp.float32)
                                       * n.astype(jnp.float32))
        pltpu.async_copy(sum_st.at[pl.ds(gi, 1)], sum_hbm.at[pl.ds(g, 1)], osem)
        pltpu.async_copy(max_st.at[pl.ds(gi, 1)], max_hbm.at[pl.ds(g, 1)], osem)
        pltpu.async_copy(cnt_st.at[pl.ds(gi, 1)], cnt_hbm.at[pl.ds(g, 1)], osem)

    for gi in range(2):
        g = params[gi][0]
        pltpu.make_async_copy(sum_st.at[pl.ds(gi, 1)],
                              sum_hbm.at[pl.ds(g, 1)], osem).wait()
        pltpu.make_async_copy(max_st.at[pl.ds(gi, 1)],
                              max_hbm.at[pl.ds(g, 1)], osem).wait()
        pltpu.make_async_copy(cnt_st.at[pl.ds(gi, 1)],
                              cnt_hbm.at[pl.ds(g, 1)], osem).wait()


def _sc_pool(h, starts128):
    mesh = plsc.VectorSubcoreMesh(core_axis_name="c", subcore_axis_name="s")
    f = pl.kernel(
        _sc_pool_body,
        mesh=mesh,
        out_type=[
            jax.ShapeDtypeStruct((NUM_GRAPHS, D), jnp.float32),
            jax.ShapeDtypeStruct((NUM_GRAPHS, D), jnp.float32),
            jax.ShapeDtypeStruct((NUM_GRAPHS, LANES), jnp.float32),
        ],
        scratch_types=[
            pltpu.VMEM((1, 128), jnp.int32),
            pltpu.VMEM((CHUNK, D), jnp.float32),
            pltpu.VMEM((CHUNK, D), jnp.float32),
            pltpu.VMEM((CHUNK, D), jnp.float32),
            pltpu.VMEM((CHUNK, D), jnp.float32),
            pltpu.VMEM((2, D), jnp.float32),
            pltpu.VMEM((2, D), jnp.float32),
            pltpu.VMEM((2, LANES), jnp.float32),
            pltpu.SemaphoreType.DMA,
            pltpu.SemaphoreType.DMA,
            pltpu.SemaphoreType.DMA,
            pltpu.SemaphoreType.DMA,
            pltpu.SemaphoreType.DMA,
        ],
    )
    return f(h, starts128)


def _mlp_body(cnt_ref, sum_ref, max_ref, w1_ref, b1_ref, w2_ref, b2_ref,
              out_ref):
    cnt = jnp.maximum(cnt_ref[:, 0:1], 1.0)
    h_sum = sum_ref[...]
    h_max = max_ref[...]
    h_mean = h_sum / cnt
    x = jnp.concatenate([h_sum, h_mean, h_max], axis=1)
    y = lax.dot_general(x, w1_ref[...], (((1,), (1,)), ((), ())),
                        preferred_element_type=jnp.float32) + b1_ref[...]
    y = 0.5 * y * (1.0 + lax.erf(y * 0.7071067811865476))
    out_ref[...] = lax.dot_general(y, w2_ref[...], (((1,), (1,)), ((), ())),
                                   preferred_element_type=jnp.float32) + b2_ref[...]


def _mlp(cnt, h_sum, h_max, W1, b1, W2, b2):
    return pl.pallas_call(
        _mlp_body,
        out_shape=jax.ShapeDtypeStruct((NUM_GRAPHS, D), jnp.float32),
    )(cnt, h_sum, h_max, W1, b1.reshape(1, D), W2, b2.reshape(1, D))


def kernel(h, batch, W1, b1, W2, b2):
    starts128 = _starts(batch)
    h_sum, h_max, cnt = _sc_pool(h, starts128)
    return _mlp(cnt, h_sum, h_max, W1, b1, W2, b2)
